# baseline (speedup 1.0000x reference)
import functools
import numpy as np
import jax
import jax.numpy as jnp
from jax import lax
from jax.experimental import pallas as pl
from jax.experimental.pallas import tpu as pltpu
from jax.experimental.pallas import tpu_sc as plsc

_NUM_CLASSES = 19
_NCLS_PAD = 24
_CH = 256
_Q = 64 * 128
_T = 16
_KS = tuple(float(k) * 0.5 for k in range(-6, 7))
_NK = len(_KS)
_TGT_NP = np.exp(-0.5 * np.asarray(_KS, np.float64) ** 2)
_TGT_NP = (_TGT_NP / _TGT_NP.sum()).astype(np.float32).reshape(1, _NK)
_CH_BLK = 128
_SUB = 8
_GRP = _CH_BLK // _SUB
_N_CHB = _CH // _CH_BLK
_MIN_N = 1000.0
_LOG2E = float(np.log2(np.e))
_SC_CORES = 2
_SC_TILES = 32
_SC_L = 16
_QT = _Q // _SC_TILES
_LPT = _T * _QT


_HPT = _NCLS_PAD * _QT


def _sc_hist_kernel(lab_hbm, w_hbm, lab_v, hist_v):
    wid = lax.axis_index("s") * _SC_CORES + lax.axis_index("c")
    pltpu.sync_copy(lab_hbm.at[wid], lab_v)

    def _zero(i, carry):
        hist_v[pl.ds(i * _SC_L, _SC_L)] = jnp.zeros((_SC_L,), jnp.float32)
        return carry

    lax.fori_loop(0, _HPT // _SC_L, _zero, 0)
    ones = jnp.full((_SC_L,), 1.0, jnp.float32)
    lanes = lax.iota(jnp.int32, _SC_L)

    def _scatter(i, carry):
        cls = lab_v[pl.ds(i * _SC_L, _SC_L)]
        idx = cls * _QT + (i % (_QT // _SC_L)) * _SC_L + lanes
        plsc.addupdate_scatter(hist_v, [idx], ones)
        return carry

    lax.fori_loop(0, _LPT // _SC_L, _scatter, 0)
    pltpu.sync_copy(hist_v, w_hbm.at[wid])


def _sc_count_hist(lab_sc):
    mesh = plsc.VectorSubcoreMesh(core_axis_name="c", subcore_axis_name="s")
    w_tiles = pl.kernel(
        _sc_hist_kernel,
        out_type=jax.ShapeDtypeStruct((_SC_TILES, _HPT), jnp.float32),
        mesh=mesh,
        scratch_types=[
            pltpu.VMEM((_LPT,), jnp.int32),
            pltpu.VMEM((_HPT,), jnp.float32),
        ],
        compiler_params=pltpu.CompilerParams(needs_layout_passes=False),
    )(lab_sc)
    return w_tiles.reshape(_SC_TILES, _NCLS_PAD, _QT).transpose(
        1, 0, 2).reshape(_NCLS_PAD, _Q)


def _prep_kernel(w_in_ref, feat_ref, w8_ref, m1_ref, m2_ref, act_ref):
    b = pl.program_id(0)

    @pl.when(b == 0)
    def _prologue():
        n_all = jnp.sum(w_in_ref[...], axis=1, keepdims=True)
        act_ref[0, 0] = jnp.sum((n_all > 0.0).astype(jnp.float32))
        for cls in range(1, _NUM_CLASSES):
            w8_ref[cls - 1, :, :] = jnp.broadcast_to(
                w_in_ref[cls, :], (_SUB, _Q))

    f = feat_ref[...]
    nt = (((1,), (1,)), ((), ()))
    m1_ref[...] = jax.lax.dot_general(
        f, w_in_ref[...], nt, preferred_element_type=jnp.float32)
    m2_ref[...] = jax.lax.dot_general(
        f * f, w_in_ref[...], nt, preferred_element_type=jnp.float32)


def _main_kernel(w8_ref, feat_ref, m1_ref, m2_ref, act_ref, out_ref, acc_ref):
    b = pl.program_id(0)
    c = pl.program_id(1)
    cls = c + 1

    @pl.when(jnp.logical_and(b == 0, c == 0))
    def _init():
        acc_ref[0, 0] = 0.0

    w8 = w8_ref[0]
    n = jnp.sum(w8) * (1.0 / _SUB)
    nf = jnp.maximum(n, 1.0)
    lane = jax.lax.broadcasted_iota(jnp.int32, (_CH_BLK, _NCLS_PAD), 1)
    sel = (lane == cls).astype(jnp.float32)
    m1c = jnp.sum(m1_ref[...] * sel, axis=1, keepdims=True)
    m2c = jnp.sum(m2_ref[...] * sel, axis=1, keepdims=True)
    miu = m1c / nf
    var = m2c / nf - miu * miu + 1e-12
    std = jnp.sqrt(var)
    cvar = (-0.5 * 25.0) / var
    scale = jnp.sqrt(-cvar * _LOG2E)
    miu3 = miu.reshape(_GRP, _SUB, 1)
    std3 = std.reshape(_GRP, _SUB, 1)
    s3 = scale.reshape(_GRP, _SUB, 1)
    log2w = (jnp.log(w8) * _LOG2E)[None]
    f3 = feat_ref[...].reshape(_GRP, _SUB, _Q)
    fs = f3 * s3
    cols = []
    for k in _KS:
        bvs = (miu3 + k * std3) * s3
        u = bvs - fs
        e = jnp.exp2(log2w - u * u)
        cols.append(jnp.sum(e, axis=2, keepdims=True))
    s_vals = jnp.concatenate(cols, axis=2)
    inv_rs = 1.0 / jnp.sum(s_vals, axis=2, keepdims=True)
    partial = 0.0
    for i in range(_NK):
        d = cols[i] * inv_rs - float(_TGT_NP[0, i])
        ad = jnp.abs(d)
        partial += jnp.sum(jnp.where(ad < 1.0, 0.5 * d * d, ad - 0.5))
    contrib = jnp.where(n >= _MIN_N, partial / (_CH * _NK), 0.0)
    acc_ref[0, 0] = acc_ref[0, 0] + contrib

    @pl.when(jnp.logical_and(b == _N_CHB - 1, c == _NUM_CLASSES - 2))
    def _epilogue():
        out_ref[0, 0] = acc_ref[0, 0] / act_ref[0, 0]


def kernel(feature, label):
    feat = feature.reshape(_CH, _Q)
    lab = label.reshape(256, 512).astype(jnp.int32)
    lab16 = lab.reshape(64, 4, 128, 4).transpose(1, 3, 0, 2).reshape(_T, _Q)
    lab_sc = lab16.reshape(_T, _SC_TILES, _QT).transpose(1, 0, 2).reshape(
        _SC_TILES, _LPT)
    w24 = _sc_count_hist(lab_sc)
    w8_bc, m1, m2, act = pl.pallas_call(
        _prep_kernel,
        grid=(_N_CHB,),
        in_specs=[
            pl.BlockSpec((_NCLS_PAD, _Q), lambda b: (0, 0)),
            pl.BlockSpec((_CH_BLK, _Q), lambda b: (b, 0)),
        ],
        out_specs=[
            pl.BlockSpec((_NUM_CLASSES - 1, _SUB, _Q), lambda b: (0, 0, 0)),
            pl.BlockSpec((_CH_BLK, _NCLS_PAD), lambda b: (b, 0)),
            pl.BlockSpec((_CH_BLK, _NCLS_PAD), lambda b: (b, 0)),
            pl.BlockSpec(memory_space=pltpu.SMEM),
        ],
        out_shape=[
            jax.ShapeDtypeStruct((_NUM_CLASSES - 1, _SUB, _Q), jnp.float32),
            jax.ShapeDtypeStruct((_CH, _NCLS_PAD), jnp.float32),
            jax.ShapeDtypeStruct((_CH, _NCLS_PAD), jnp.float32),
            jax.ShapeDtypeStruct((1, 1), jnp.float32),
        ],
    )(w24, feat)
    out = pl.pallas_call(
        _main_kernel,
        grid=(_N_CHB, _NUM_CLASSES - 1),
        in_specs=[
            pl.BlockSpec((1, _SUB, _Q), lambda b, c: (c, 0, 0)),
            pl.BlockSpec((_CH_BLK, _Q), lambda b, c: (b, 0)),
            pl.BlockSpec((_CH_BLK, _NCLS_PAD), lambda b, c: (b, 0)),
            pl.BlockSpec((_CH_BLK, _NCLS_PAD), lambda b, c: (b, 0)),
            pl.BlockSpec(memory_space=pltpu.SMEM),
        ],
        out_specs=pl.BlockSpec(memory_space=pltpu.SMEM),
        out_shape=jax.ShapeDtypeStruct((1, 1), jnp.float32),
        scratch_shapes=[pltpu.SMEM((1, 1), jnp.float32)],
    )(w8_bc, feat, m1, m2, act)
    return out.reshape(())


# speedup vs baseline: 1.0680x; 1.0680x over previous
import functools
import numpy as np
import jax
import jax.numpy as jnp
from jax import lax
from jax.experimental import pallas as pl
from jax.experimental.pallas import tpu as pltpu
from jax.experimental.pallas import tpu_sc as plsc

_NUM_CLASSES = 19
_NCLS_PAD = 24
_CH = 256
_Q = 64 * 128
_T = 16
_KS = tuple(float(k) * 0.5 for k in range(-6, 7))
_NK = len(_KS)
_TGT_NP = np.exp(-0.5 * np.asarray(_KS, np.float64) ** 2)
_TGT_NP = (_TGT_NP / _TGT_NP.sum()).astype(np.float32).reshape(1, _NK)
_CH_BLK = 128
_SUB = 8
_GRP = _CH_BLK // _SUB
_N_CHB = _CH // _CH_BLK
_MIN_N = 1000.0
_LOG2E = float(np.log2(np.e))
_SC_CORES = 2
_SC_TILES = 32
_SC_L = 16
_QT = _Q // _SC_TILES
_LPT = _T * _QT


_HPT = _NCLS_PAD * _QT


def _sc_hist_kernel(lab_hbm, w_hbm, lab_v, hist_v):
    wid = lax.axis_index("s") * _SC_CORES + lax.axis_index("c")
    pltpu.sync_copy(lab_hbm.at[wid], lab_v)

    def _zero(i, carry):
        hist_v[pl.ds(i * _SC_L, _SC_L)] = jnp.zeros((_SC_L,), jnp.float32)
        return carry

    lax.fori_loop(0, _HPT // _SC_L, _zero, 0)
    ones = jnp.full((_SC_L,), 1.0, jnp.float32)
    lanes = lax.iota(jnp.int32, _SC_L)

    def _scatter(m, carry):
        i_local = m // 32
        joff = (m // 8) % 4
        g = m % 8
        cls = plsc.load_gather(
            lab_v, [i_local * 512 + joff + 4 * (g * _SC_L + lanes)])
        q = (i_local // 4) * 128 + g * _SC_L + lanes
        plsc.addupdate_scatter(hist_v, [cls * _QT + q], ones)
        return carry

    lax.fori_loop(0, _LPT // _SC_L, _scatter, 0)
    pltpu.sync_copy(hist_v, w_hbm.at[wid])


def _sc_count_hist(lab_sc):
    mesh = plsc.VectorSubcoreMesh(core_axis_name="c", subcore_axis_name="s")
    w_tiles = pl.kernel(
        _sc_hist_kernel,
        out_type=jax.ShapeDtypeStruct((_SC_TILES, _HPT), jnp.float32),
        mesh=mesh,
        scratch_types=[
            pltpu.VMEM((_LPT,), jnp.int32),
            pltpu.VMEM((_HPT,), jnp.float32),
        ],
        compiler_params=pltpu.CompilerParams(needs_layout_passes=False),
    )(lab_sc)
    return w_tiles.reshape(_SC_TILES, _NCLS_PAD, _QT).transpose(
        1, 0, 2).reshape(_NCLS_PAD, _Q)


def _prep_kernel(w_in_ref, feat_ref, w8_ref, m1_ref, m2_ref, act_ref):
    b = pl.program_id(0)

    @pl.when(b == 0)
    def _prologue():
        n_all = jnp.sum(w_in_ref[...], axis=1, keepdims=True)
        act_ref[0, 0] = jnp.sum((n_all > 0.0).astype(jnp.float32))
        for cls in range(1, _NUM_CLASSES):
            w8_ref[cls - 1, :, :] = jnp.broadcast_to(
                w_in_ref[cls, :], (_SUB, _Q))

    f = feat_ref[...]
    nt = (((1,), (1,)), ((), ()))
    m1_ref[...] = jax.lax.dot_general(
        f, w_in_ref[...], nt, preferred_element_type=jnp.float32)
    m2_ref[...] = jax.lax.dot_general(
        f * f, w_in_ref[...], nt, preferred_element_type=jnp.float32)


def _main_kernel(w8_ref, feat_ref, m1_ref, m2_ref, act_ref, out_ref, acc_ref):
    b = pl.program_id(0)
    c = pl.program_id(1)
    cls = c + 1

    @pl.when(jnp.logical_and(b == 0, c == 0))
    def _init():
        acc_ref[0, 0] = 0.0

    w8 = w8_ref[0]
    n = jnp.sum(w8) * (1.0 / _SUB)
    nf = jnp.maximum(n, 1.0)
    lane = jax.lax.broadcasted_iota(jnp.int32, (_CH_BLK, _NCLS_PAD), 1)
    sel = (lane == cls).astype(jnp.float32)
    m1c = jnp.sum(m1_ref[...] * sel, axis=1, keepdims=True)
    m2c = jnp.sum(m2_ref[...] * sel, axis=1, keepdims=True)
    miu = m1c / nf
    var = m2c / nf - miu * miu + 1e-12
    std = jnp.sqrt(var)
    cvar = (-0.5 * 25.0) / var
    scale = jnp.sqrt(-cvar * _LOG2E)
    miu3 = miu.reshape(_GRP, _SUB, 1)
    std3 = std.reshape(_GRP, _SUB, 1)
    s3 = scale.reshape(_GRP, _SUB, 1)
    log2w = (jnp.log(w8) * _LOG2E)[None]
    f3 = feat_ref[...].reshape(_GRP, _SUB, _Q)
    fs = f3 * s3
    cols = []
    for k in _KS:
        bvs = (miu3 + k * std3) * s3
        u = bvs - fs
        e = jnp.exp2(log2w - u * u)
        cols.append(jnp.sum(e, axis=2, keepdims=True))
    s_vals = jnp.concatenate(cols, axis=2)
    inv_rs = 1.0 / jnp.sum(s_vals, axis=2, keepdims=True)
    partial = 0.0
    for i in range(_NK):
        d = cols[i] * inv_rs - float(_TGT_NP[0, i])
        ad = jnp.abs(d)
        partial += jnp.sum(jnp.where(ad < 1.0, 0.5 * d * d, ad - 0.5))
    contrib = jnp.where(n >= _MIN_N, partial / (_CH * _NK), 0.0)
    acc_ref[0, 0] = acc_ref[0, 0] + contrib

    @pl.when(jnp.logical_and(b == _N_CHB - 1, c == _NUM_CLASSES - 2))
    def _epilogue():
        out_ref[0, 0] = acc_ref[0, 0] / act_ref[0, 0]


def kernel(feature, label):
    feat = feature.reshape(_CH, _Q)
    lab_sc = label.astype(jnp.int32).reshape(_SC_TILES, _LPT)
    w24 = _sc_count_hist(lab_sc)
    w8_bc, m1, m2, act = pl.pallas_call(
        _prep_kernel,
        grid=(_N_CHB,),
        in_specs=[
            pl.BlockSpec((_NCLS_PAD, _Q), lambda b: (0, 0)),
            pl.BlockSpec((_CH_BLK, _Q), lambda b: (b, 0)),
        ],
        out_specs=[
            pl.BlockSpec((_NUM_CLASSES - 1, _SUB, _Q), lambda b: (0, 0, 0)),
            pl.BlockSpec((_CH_BLK, _NCLS_PAD), lambda b: (b, 0)),
            pl.BlockSpec((_CH_BLK, _NCLS_PAD), lambda b: (b, 0)),
            pl.BlockSpec(memory_space=pltpu.SMEM),
        ],
        out_shape=[
            jax.ShapeDtypeStruct((_NUM_CLASSES - 1, _SUB, _Q), jnp.float32),
            jax.ShapeDtypeStruct((_CH, _NCLS_PAD), jnp.float32),
            jax.ShapeDtypeStruct((_CH, _NCLS_PAD), jnp.float32),
            jax.ShapeDtypeStruct((1, 1), jnp.float32),
        ],
    )(w24, feat)
    out = pl.pallas_call(
        _main_kernel,
        grid=(_N_CHB, _NUM_CLASSES - 1),
        in_specs=[
            pl.BlockSpec((1, _SUB, _Q), lambda b, c: (c, 0, 0)),
            pl.BlockSpec((_CH_BLK, _Q), lambda b, c: (b, 0)),
            pl.BlockSpec((_CH_BLK, _NCLS_PAD), lambda b, c: (b, 0)),
            pl.BlockSpec((_CH_BLK, _NCLS_PAD), lambda b, c: (b, 0)),
            pl.BlockSpec(memory_space=pltpu.SMEM),
        ],
        out_specs=pl.BlockSpec(memory_space=pltpu.SMEM),
        out_shape=jax.ShapeDtypeStruct((1, 1), jnp.float32),
        scratch_shapes=[pltpu.SMEM((1, 1), jnp.float32)],
    )(w8_bc, feat, m1, m2, act)
    return out.reshape(())


# speedup vs baseline: 1.0696x; 1.0015x over previous
import functools
import numpy as np
import jax
import jax.numpy as jnp
from jax import lax
from jax.experimental import pallas as pl
from jax.experimental.pallas import tpu as pltpu
from jax.experimental.pallas import tpu_sc as plsc

_NUM_CLASSES = 19
_NCLS_PAD = 24
_CH = 256
_Q = 64 * 128
_T = 16
_KS = tuple(float(k) * 0.5 for k in range(-6, 7))
_NK = len(_KS)
_TGT_NP = np.exp(-0.5 * np.asarray(_KS, np.float64) ** 2)
_TGT_NP = (_TGT_NP / _TGT_NP.sum()).astype(np.float32).reshape(1, _NK)
_CH_BLK = 128
_SUB = 8
_GRP = _CH_BLK // _SUB
_N_CHB = _CH // _CH_BLK
_MIN_N = 1000.0
_LOG2E = float(np.log2(np.e))
_SC_CORES = 2
_SC_TILES = 32
_SC_L = 16
_QT = _Q // _SC_TILES
_LPT = _T * _QT


_HPT = _NCLS_PAD * _QT


def _sc_hist_kernel(lab_hbm, w_hbm, lab_v, hist_v, sem):
    wid = lax.axis_index("s") * _SC_CORES + lax.axis_index("c")
    pltpu.sync_copy(lab_hbm.at[wid], lab_v)

    def _zero(i, carry):
        hist_v[pl.ds(i * _SC_L, _SC_L)] = jnp.zeros((_SC_L,), jnp.float32)
        return carry

    lax.fori_loop(0, _HPT // _SC_L, _zero, 0)
    ones = jnp.full((_SC_L,), 1.0, jnp.float32)
    lanes = lax.iota(jnp.int32, _SC_L)

    def _scatter(m, carry):
        i_local = m // 32
        joff = (m // 8) % 4
        g = m % 8
        cls = plsc.load_gather(
            lab_v, [i_local * 512 + joff + 4 * (g * _SC_L + lanes)])
        q = (i_local // 4) * 128 + g * _SC_L + lanes
        plsc.addupdate_scatter(hist_v, [cls * _QT + q], ones)
        return carry

    lax.fori_loop(0, _LPT // _SC_L, _scatter, 0)
    copies = [
        pltpu.async_copy(
            hist_v.at[pl.ds(r * _QT, _QT)],
            w_hbm.at[r, pl.ds(wid * _QT, _QT)], sem)
        for r in range(_NCLS_PAD)
    ]
    for c in copies:
        c.wait()


def _sc_count_hist(lab_sc):
    mesh = plsc.VectorSubcoreMesh(core_axis_name="c", subcore_axis_name="s")
    return pl.kernel(
        _sc_hist_kernel,
        out_type=jax.ShapeDtypeStruct((_NCLS_PAD, _Q), jnp.float32),
        mesh=mesh,
        scratch_types=[
            pltpu.VMEM((_LPT,), jnp.int32),
            pltpu.VMEM((_HPT,), jnp.float32),
            pltpu.SemaphoreType.DMA,
        ],
        compiler_params=pltpu.CompilerParams(needs_layout_passes=False),
    )(lab_sc)


def _prep_kernel(w_in_ref, feat_ref, w8_ref, m1_ref, m2_ref, act_ref):
    b = pl.program_id(0)

    @pl.when(b == 0)
    def _prologue():
        n_all = jnp.sum(w_in_ref[...], axis=1, keepdims=True)
        act_ref[0, 0] = jnp.sum((n_all > 0.0).astype(jnp.float32))
        for cls in range(1, _NUM_CLASSES):
            w8_ref[cls - 1, :, :] = jnp.broadcast_to(
                w_in_ref[cls, :], (_SUB, _Q))

    f = feat_ref[...]
    nt = (((1,), (1,)), ((), ()))
    m1_ref[...] = jax.lax.dot_general(
        f, w_in_ref[...], nt, preferred_element_type=jnp.float32)
    m2_ref[...] = jax.lax.dot_general(
        f * f, w_in_ref[...], nt, preferred_element_type=jnp.float32)


def _main_kernel(w8_ref, feat_ref, m1_ref, m2_ref, act_ref, out_ref, acc_ref):
    b = pl.program_id(0)
    c = pl.program_id(1)
    cls = c + 1

    @pl.when(jnp.logical_and(b == 0, c == 0))
    def _init():
        acc_ref[0, 0] = 0.0

    w8 = w8_ref[0]
    n = jnp.sum(w8) * (1.0 / _SUB)
    nf = jnp.maximum(n, 1.0)
    lane = jax.lax.broadcasted_iota(jnp.int32, (_CH_BLK, _NCLS_PAD), 1)
    sel = (lane == cls).astype(jnp.float32)
    m1c = jnp.sum(m1_ref[...] * sel, axis=1, keepdims=True)
    m2c = jnp.sum(m2_ref[...] * sel, axis=1, keepdims=True)
    miu = m1c / nf
    var = m2c / nf - miu * miu + 1e-12
    std = jnp.sqrt(var)
    cvar = (-0.5 * 25.0) / var
    scale = jnp.sqrt(-cvar * _LOG2E)
    miu3 = miu.reshape(_GRP, _SUB, 1)
    std3 = std.reshape(_GRP, _SUB, 1)
    s3 = scale.reshape(_GRP, _SUB, 1)
    log2w = (jnp.log(w8) * _LOG2E)[None]
    f3 = feat_ref[...].reshape(_GRP, _SUB, _Q)
    fs = f3 * s3
    cols = []
    for k in _KS:
        bvs = (miu3 + k * std3) * s3
        u = bvs - fs
        e = jnp.exp2(log2w - u * u)
        cols.append(jnp.sum(e, axis=2, keepdims=True))
    s_vals = jnp.concatenate(cols, axis=2)
    inv_rs = 1.0 / jnp.sum(s_vals, axis=2, keepdims=True)
    partial = 0.0
    for i in range(_NK):
        d = cols[i] * inv_rs - float(_TGT_NP[0, i])
        ad = jnp.abs(d)
        partial += jnp.sum(jnp.where(ad < 1.0, 0.5 * d * d, ad - 0.5))
    contrib = jnp.where(n >= _MIN_N, partial / (_CH * _NK), 0.0)
    acc_ref[0, 0] = acc_ref[0, 0] + contrib

    @pl.when(jnp.logical_and(b == _N_CHB - 1, c == _NUM_CLASSES - 2))
    def _epilogue():
        out_ref[0, 0] = acc_ref[0, 0] / act_ref[0, 0]


def kernel(feature, label):
    feat = feature.reshape(_CH, _Q)
    lab_sc = label.astype(jnp.int32).reshape(_SC_TILES, _LPT)
    w24 = _sc_count_hist(lab_sc)
    w8_bc, m1, m2, act = pl.pallas_call(
        _prep_kernel,
        grid=(_N_CHB,),
        in_specs=[
            pl.BlockSpec((_NCLS_PAD, _Q), lambda b: (0, 0)),
            pl.BlockSpec((_CH_BLK, _Q), lambda b: (b, 0)),
        ],
        out_specs=[
            pl.BlockSpec((_NUM_CLASSES - 1, _SUB, _Q), lambda b: (0, 0, 0)),
            pl.BlockSpec((_CH_BLK, _NCLS_PAD), lambda b: (b, 0)),
            pl.BlockSpec((_CH_BLK, _NCLS_PAD), lambda b: (b, 0)),
            pl.BlockSpec(memory_space=pltpu.SMEM),
        ],
        out_shape=[
            jax.ShapeDtypeStruct((_NUM_CLASSES - 1, _SUB, _Q), jnp.float32),
            jax.ShapeDtypeStruct((_CH, _NCLS_PAD), jnp.float32),
            jax.ShapeDtypeStruct((_CH, _NCLS_PAD), jnp.float32),
            jax.ShapeDtypeStruct((1, 1), jnp.float32),
        ],
    )(w24, feat)
    out = pl.pallas_call(
        _main_kernel,
        grid=(_N_CHB, _NUM_CLASSES - 1),
        in_specs=[
            pl.BlockSpec((1, _SUB, _Q), lambda b, c: (c, 0, 0)),
            pl.BlockSpec((_CH_BLK, _Q), lambda b, c: (b, 0)),
            pl.BlockSpec((_CH_BLK, _NCLS_PAD), lambda b, c: (b, 0)),
            pl.BlockSpec((_CH_BLK, _NCLS_PAD), lambda b, c: (b, 0)),
            pl.BlockSpec(memory_space=pltpu.SMEM),
        ],
        out_specs=pl.BlockSpec(memory_space=pltpu.SMEM),
        out_shape=jax.ShapeDtypeStruct((1, 1), jnp.float32),
        scratch_shapes=[pltpu.SMEM((1, 1), jnp.float32)],
    )(w8_bc, feat, m1, m2, act)
    return out.reshape(())


# speedup vs baseline: 1.0836x; 1.0131x over previous
import functools
import numpy as np
import jax
import jax.numpy as jnp
from jax import lax
from jax.experimental import pallas as pl
from jax.experimental.pallas import tpu as pltpu
from jax.experimental.pallas import tpu_sc as plsc

_NUM_CLASSES = 19
_NCLS_PAD = 24
_CH = 256
_Q = 64 * 128
_T = 16
_KS = tuple(float(k) * 0.5 for k in range(-6, 7))
_NK = len(_KS)
_TGT_NP = np.exp(-0.5 * np.asarray(_KS, np.float64) ** 2)
_TGT_NP = (_TGT_NP / _TGT_NP.sum()).astype(np.float32).reshape(1, _NK)
_CH_BLK = 256
_SUB = 8
_GRP = _CH_BLK // _SUB
_N_CHB = _CH // _CH_BLK
_MIN_N = 1000.0
_LOG2E = float(np.log2(np.e))
_SC_CORES = 2
_SC_TILES = 32
_SC_L = 16
_QT = _Q // _SC_TILES
_LPT = _T * _QT


_HPT = _NCLS_PAD * _QT


def _sc_hist_kernel(lab_hbm, w_hbm, lab_v, hist_v, sem):
    wid = lax.axis_index("s") * _SC_CORES + lax.axis_index("c")
    pltpu.sync_copy(lab_hbm.at[wid], lab_v)

    def _zero(i, carry):
        hist_v[pl.ds(i * _SC_L, _SC_L)] = jnp.zeros((_SC_L,), jnp.float32)
        return carry

    lax.fori_loop(0, _HPT // _SC_L, _zero, 0)
    ones = jnp.full((_SC_L,), 1.0, jnp.float32)
    lanes = lax.iota(jnp.int32, _SC_L)

    def _scatter(m, carry):
        i_local = m // 32
        joff = (m // 8) % 4
        g = m % 8
        cls = plsc.load_gather(
            lab_v, [i_local * 512 + joff + 4 * (g * _SC_L + lanes)])
        q = (i_local // 4) * 128 + g * _SC_L + lanes
        plsc.addupdate_scatter(hist_v, [cls * _QT + q], ones)
        return carry

    lax.fori_loop(0, _LPT // _SC_L, _scatter, 0)
    copies = [
        pltpu.async_copy(
            hist_v.at[pl.ds(r * _QT, _QT)],
            w_hbm.at[r, pl.ds(wid * _QT, _QT)], sem)
        for r in range(_NCLS_PAD)
    ]
    for c in copies:
        c.wait()


def _sc_count_hist(lab_sc):
    mesh = plsc.VectorSubcoreMesh(core_axis_name="c", subcore_axis_name="s")
    return pl.kernel(
        _sc_hist_kernel,
        out_type=jax.ShapeDtypeStruct((_NCLS_PAD, _Q), jnp.float32),
        mesh=mesh,
        scratch_types=[
            pltpu.VMEM((_LPT,), jnp.int32),
            pltpu.VMEM((_HPT,), jnp.float32),
            pltpu.SemaphoreType.DMA,
        ],
        compiler_params=pltpu.CompilerParams(needs_layout_passes=False),
    )(lab_sc)


def _prep_kernel(w_in_ref, feat_ref, w8_ref, m1_ref, m2_ref, act_ref):
    b = pl.program_id(0)

    @pl.when(b == 0)
    def _prologue():
        n_all = jnp.sum(w_in_ref[...], axis=1, keepdims=True)
        act_ref[0, 0] = jnp.sum((n_all > 0.0).astype(jnp.float32))
        for cls in range(1, _NUM_CLASSES):
            w8_ref[cls - 1, :, :] = jnp.broadcast_to(
                w_in_ref[cls, :], (_SUB, _Q))

    f = feat_ref[...]
    nt = (((1,), (1,)), ((), ()))
    m1_ref[...] = jax.lax.dot_general(
        f, w_in_ref[...], nt, preferred_element_type=jnp.float32)
    m2_ref[...] = jax.lax.dot_general(
        f * f, w_in_ref[...], nt, preferred_element_type=jnp.float32)


def _main_kernel(w8_ref, feat_ref, m1_ref, m2_ref, act_ref, out_ref, acc_ref):
    b = pl.program_id(0)
    c = pl.program_id(1)
    cls = c + 1

    @pl.when(jnp.logical_and(b == 0, c == 0))
    def _init():
        acc_ref[0, 0] = 0.0

    w8 = w8_ref[0]
    n = jnp.sum(w8) * (1.0 / _SUB)
    nf = jnp.maximum(n, 1.0)
    lane = jax.lax.broadcasted_iota(jnp.int32, (_CH_BLK, _NCLS_PAD), 1)
    sel = (lane == cls).astype(jnp.float32)
    m1c = jnp.sum(m1_ref[...] * sel, axis=1, keepdims=True)
    m2c = jnp.sum(m2_ref[...] * sel, axis=1, keepdims=True)
    miu = m1c / nf
    var = m2c / nf - miu * miu + 1e-12
    std = jnp.sqrt(var)
    cvar = (-0.5 * 25.0) / var
    scale = jnp.sqrt(-cvar * _LOG2E)
    miu3 = miu.reshape(_GRP, _SUB, 1)
    std3 = std.reshape(_GRP, _SUB, 1)
    s3 = scale.reshape(_GRP, _SUB, 1)
    log2w = (jnp.log(w8) * _LOG2E)[None]
    f3 = feat_ref[...].reshape(_GRP, _SUB, _Q)
    fs = f3 * s3
    cols = []
    for k in _KS:
        bvs = (miu3 + k * std3) * s3
        u = bvs - fs
        e = jnp.exp2(log2w - u * u)
        cols.append(jnp.sum(e, axis=2, keepdims=True))
    s_vals = jnp.concatenate(cols, axis=2)
    inv_rs = 1.0 / jnp.sum(s_vals, axis=2, keepdims=True)
    partial = 0.0
    for i in range(_NK):
        d = cols[i] * inv_rs - float(_TGT_NP[0, i])
        ad = jnp.abs(d)
        partial += jnp.sum(jnp.where(ad < 1.0, 0.5 * d * d, ad - 0.5))
    contrib = jnp.where(n >= _MIN_N, partial / (_CH * _NK), 0.0)
    acc_ref[0, 0] = acc_ref[0, 0] + contrib

    @pl.when(jnp.logical_and(b == _N_CHB - 1, c == _NUM_CLASSES - 2))
    def _epilogue():
        out_ref[0, 0] = acc_ref[0, 0] / act_ref[0, 0]


def kernel(feature, label):
    feat = feature.reshape(_CH, _Q)
    lab_sc = label.astype(jnp.int32).reshape(_SC_TILES, _LPT)
    w24 = _sc_count_hist(lab_sc)
    w8_bc, m1, m2, act = pl.pallas_call(
        _prep_kernel,
        grid=(_N_CHB,),
        in_specs=[
            pl.BlockSpec((_NCLS_PAD, _Q), lambda b: (0, 0)),
            pl.BlockSpec((_CH_BLK, _Q), lambda b: (b, 0)),
        ],
        out_specs=[
            pl.BlockSpec((_NUM_CLASSES - 1, _SUB, _Q), lambda b: (0, 0, 0)),
            pl.BlockSpec((_CH_BLK, _NCLS_PAD), lambda b: (b, 0)),
            pl.BlockSpec((_CH_BLK, _NCLS_PAD), lambda b: (b, 0)),
            pl.BlockSpec(memory_space=pltpu.SMEM),
        ],
        out_shape=[
            jax.ShapeDtypeStruct((_NUM_CLASSES - 1, _SUB, _Q), jnp.float32),
            jax.ShapeDtypeStruct((_CH, _NCLS_PAD), jnp.float32),
            jax.ShapeDtypeStruct((_CH, _NCLS_PAD), jnp.float32),
            jax.ShapeDtypeStruct((1, 1), jnp.float32),
        ],
    )(w24, feat)
    out = pl.pallas_call(
        _main_kernel,
        grid=(_N_CHB, _NUM_CLASSES - 1),
        in_specs=[
            pl.BlockSpec((1, _SUB, _Q), lambda b, c: (c, 0, 0)),
            pl.BlockSpec((_CH_BLK, _Q), lambda b, c: (b, 0)),
            pl.BlockSpec((_CH_BLK, _NCLS_PAD), lambda b, c: (b, 0)),
            pl.BlockSpec((_CH_BLK, _NCLS_PAD), lambda b, c: (b, 0)),
            pl.BlockSpec(memory_space=pltpu.SMEM),
        ],
        out_specs=pl.BlockSpec(memory_space=pltpu.SMEM),
        out_shape=jax.ShapeDtypeStruct((1, 1), jnp.float32),
        scratch_shapes=[pltpu.SMEM((1, 1), jnp.float32)],
    )(w8_bc, feat, m1, m2, act)
    return out.reshape(())


# speedup vs baseline: 1.0838x; 1.0001x over previous
import numpy as np
import jax
import jax.numpy as jnp
from jax import lax
from jax.experimental import pallas as pl
from jax.experimental.pallas import tpu as pltpu
from jax.experimental.pallas import tpu_sc as plsc

_NUM_CLASSES = 19
_NCLS_PAD = 24
_CH = 256
_Q = 64 * 128
_T = 16
_KS = tuple(float(k) * 0.5 for k in range(-6, 7))
_NK = len(_KS)
_TGT_NP = np.exp(-0.5 * np.asarray(_KS, np.float64) ** 2)
_TGT_NP = (_TGT_NP / _TGT_NP.sum()).astype(np.float32).reshape(1, _NK)
_CH_BLK = 256
_SUB = 8
_GRP = _CH_BLK // _SUB
_N_CHB = _CH // _CH_BLK
_MIN_N = 1000.0
_LOG2E = float(np.log2(np.e))
_SC_CORES = 2
_SC_TILES = 32
_SC_L = 16
_QT = _Q // _SC_TILES
_LPT = _T * _QT


_HPT = _NCLS_PAD * _QT


def _sc_hist_kernel(lab_hbm, w_hbm, lab_v, hist_v, sem):
    wid = lax.axis_index("s") * _SC_CORES + lax.axis_index("c")
    pltpu.sync_copy(lab_hbm.at[wid], lab_v)

    def _zero(i, carry):
        hist_v[pl.ds(i * _SC_L, _SC_L)] = jnp.zeros((_SC_L,), jnp.float32)
        return carry

    lax.fori_loop(0, _HPT // _SC_L, _zero, 0)
    ones = jnp.full((_SC_L,), 1.0, jnp.float32)
    lanes = lax.iota(jnp.int32, _SC_L)

    def _scatter(m, carry):
        i_local = m // 32
        joff = (m // 8) % 4
        g = m % 8
        cls = plsc.load_gather(
            lab_v, [i_local * 512 + joff + 4 * (g * _SC_L + lanes)])
        q = (i_local // 4) * 128 + g * _SC_L + lanes
        plsc.addupdate_scatter(hist_v, [cls * _QT + q], ones)
        return carry

    lax.fori_loop(0, _LPT // _SC_L, _scatter, 0)
    copies = [
        pltpu.async_copy(
            hist_v.at[pl.ds(r * _QT, _QT)],
            w_hbm.at[r, pl.ds(wid * _QT, _QT)], sem)
        for r in range(_NCLS_PAD)
    ]
    for c in copies:
        c.wait()


def _sc_count_hist(lab_sc):
    mesh = plsc.VectorSubcoreMesh(core_axis_name="c", subcore_axis_name="s")
    return pl.kernel(
        _sc_hist_kernel,
        out_type=jax.ShapeDtypeStruct((_NCLS_PAD, _Q), jnp.float32),
        mesh=mesh,
        scratch_types=[
            pltpu.VMEM((_LPT,), jnp.int32),
            pltpu.VMEM((_HPT,), jnp.float32),
            pltpu.SemaphoreType.DMA,
        ],
        compiler_params=pltpu.CompilerParams(needs_layout_passes=False),
    )(lab_sc)


def _prep_kernel(w_in_ref, feat_ref, w8_ref, m1_ref, m2_ref, act_ref):
    b = pl.program_id(0)

    @pl.when(b == 0)
    def _prologue():
        n_all = jnp.sum(w_in_ref[...], axis=1, keepdims=True)
        act_ref[0, 0] = jnp.sum((n_all > 0.0).astype(jnp.float32))
        for cls in range(1, _NUM_CLASSES):
            w8_ref[cls - 1, :, :] = jnp.broadcast_to(
                w_in_ref[cls, :], (_SUB, _Q))

    f = feat_ref[...]
    nt = (((1,), (1,)), ((), ()))
    m1_ref[...] = jax.lax.dot_general(
        f, w_in_ref[...], nt, preferred_element_type=jnp.float32)
    m2_ref[...] = jax.lax.dot_general(
        f * f, w_in_ref[...], nt, preferred_element_type=jnp.float32)


def _main_kernel(w8_ref, feat_ref, m1_ref, m2_ref, act_ref, out_ref, acc_ref):
    b = pl.program_id(0)
    c = pl.program_id(1)
    cls = c + 1

    @pl.when(jnp.logical_and(b == 0, c == 0))
    def _init():
        acc_ref[0, 0] = 0.0

    w8 = w8_ref[0]
    n = jnp.sum(w8) * (1.0 / _SUB)
    nf = jnp.maximum(n, 1.0)
    lane = jax.lax.broadcasted_iota(jnp.int32, (_CH_BLK, _NCLS_PAD), 1)
    sel = (lane == cls).astype(jnp.float32)
    m1c = jnp.sum(m1_ref[...] * sel, axis=1, keepdims=True)
    m2c = jnp.sum(m2_ref[...] * sel, axis=1, keepdims=True)
    miu = m1c / nf
    var = m2c / nf - miu * miu + 1e-12
    std = jnp.sqrt(var)
    cvar = (-0.5 * 25.0) / var
    scale = jnp.sqrt(-cvar * _LOG2E)
    miu3 = miu.reshape(_GRP, _SUB, 1)
    std3 = std.reshape(_GRP, _SUB, 1)
    s3 = scale.reshape(_GRP, _SUB, 1)
    log2w = (jnp.log(w8) * _LOG2E)[None]
    f3 = feat_ref[...].reshape(_GRP, _SUB, _Q)
    fs = f3 * s3
    cols = []
    for k in _KS:
        bvs = (miu3 + k * std3) * s3
        u = bvs - fs
        e = jnp.exp2(log2w - u * u)
        cols.append(jnp.sum(e, axis=2, keepdims=True))
    s_vals = jnp.concatenate(cols, axis=2)
    inv_rs = 1.0 / jnp.sum(s_vals, axis=2, keepdims=True)
    partial = 0.0
    for i in range(_NK):
        d = cols[i] * inv_rs - float(_TGT_NP[0, i])
        ad = jnp.abs(d)
        partial += jnp.sum(jnp.where(ad < 1.0, 0.5 * d * d, ad - 0.5))
    contrib = jnp.where(n >= _MIN_N, partial / (_CH * _NK), 0.0)
    acc_ref[0, 0] = acc_ref[0, 0] + contrib

    @pl.when(jnp.logical_and(b == _N_CHB - 1, c == _NUM_CLASSES - 2))
    def _epilogue():
        out_ref[0, 0] = acc_ref[0, 0] / act_ref[0, 0]


def kernel(feature, label):
    feat = feature.reshape(_CH, _Q)
    lab_sc = label.astype(jnp.int32).reshape(_SC_TILES, _LPT)
    w24 = _sc_count_hist(lab_sc)
    w8_bc, m1, m2, act = pl.pallas_call(
        _prep_kernel,
        grid=(_N_CHB,),
        in_specs=[
            pl.BlockSpec((_NCLS_PAD, _Q), lambda b: (0, 0)),
            pl.BlockSpec((_CH_BLK, _Q), lambda b: (b, 0)),
        ],
        out_specs=[
            pl.BlockSpec((_NUM_CLASSES - 1, _SUB, _Q), lambda b: (0, 0, 0)),
            pl.BlockSpec((_CH_BLK, _NCLS_PAD), lambda b: (b, 0)),
            pl.BlockSpec((_CH_BLK, _NCLS_PAD), lambda b: (b, 0)),
            pl.BlockSpec(memory_space=pltpu.SMEM),
        ],
        out_shape=[
            jax.ShapeDtypeStruct((_NUM_CLASSES - 1, _SUB, _Q), jnp.float32),
            jax.ShapeDtypeStruct((_CH, _NCLS_PAD), jnp.float32),
            jax.ShapeDtypeStruct((_CH, _NCLS_PAD), jnp.float32),
            jax.ShapeDtypeStruct((1, 1), jnp.float32),
        ],
    )(w24, feat)
    out = pl.pallas_call(
        _main_kernel,
        grid=(_N_CHB, _NUM_CLASSES - 1),
        in_specs=[
            pl.BlockSpec((1, _SUB, _Q), lambda b, c: (c, 0, 0)),
            pl.BlockSpec((_CH_BLK, _Q), lambda b, c: (b, 0)),
            pl.BlockSpec((_CH_BLK, _NCLS_PAD), lambda b, c: (b, 0)),
            pl.BlockSpec((_CH_BLK, _NCLS_PAD), lambda b, c: (b, 0)),
            pl.BlockSpec(memory_space=pltpu.SMEM),
        ],
        out_specs=pl.BlockSpec(memory_space=pltpu.SMEM),
        out_shape=jax.ShapeDtypeStruct((1, 1), jnp.float32),
        scratch_shapes=[pltpu.SMEM((1, 1), jnp.float32)],
    )(w8_bc, feat, m1, m2, act)
    return out.reshape(())
